# Initial kernel scaffold; baseline (speedup 1.0000x reference)
import functools

import jax
import jax.numpy as jnp
from jax import lax
from jax.experimental import pallas as pl
from jax.experimental.pallas import tpu as pltpu
from jax.experimental.pallas import tpu_sc as plsc

NEG = -1e30


def _bn_relu(h, g, be):
    m = jnp.mean(h, axis=0, keepdims=True)
    v = jnp.mean((h - m) ** 2, axis=0, keepdims=True)
    return jnp.maximum(g * (h - m) * lax.rsqrt(v + 1e-5) + be, 0.0)


def _mlp_body(x_ref, W1_ref, b1_ref, g1_ref, be1_ref, W2_ref, b2_ref,
              g2_ref, be2_ref, o_ref):
    h = jnp.dot(x_ref[...], W1_ref[...], preferred_element_type=jnp.float32)
    a = _bn_relu(h + b1_ref[...], g1_ref[...], be1_ref[...])
    h2 = jnp.dot(a, W2_ref[...], preferred_element_type=jnp.float32)
    o_ref[...] = _bn_relu(h2 + b2_ref[...], g2_ref[...], be2_ref[...])


def _mlp3_body(x_ref, p0_ref, p1_ref, W1_ref, b1_ref, g1_ref, be1_ref,
               W2_ref, b2_ref, g2_ref, be2_ref, o_ref):
    y = x_ref[...] + p0_ref[...] + p1_ref[...]
    h = jnp.dot(y, W1_ref[...], preferred_element_type=jnp.float32)
    a = _bn_relu(h + b1_ref[...], g1_ref[...], be1_ref[...])
    h2 = jnp.dot(a, W2_ref[...], preferred_element_type=jnp.float32)
    o_ref[...] = _bn_relu(h2 + b2_ref[...], g2_ref[...], be2_ref[...])


def _segmax(z, bcol, G):
    rows = []
    for g in range(G):
        rows.append(jnp.max(jnp.where(bcol == g, z, NEG), axis=0, keepdims=True))
    return jnp.concatenate(rows, axis=0)


def _head0_body(x0_ref, lW_ref, lb_ref, bcol_ref, z_ref, out_ref):
    G = out_ref.shape[0]
    z = jnp.dot(x0_ref[...], lW_ref[...], preferred_element_type=jnp.float32)
    z = z + lb_ref[...]
    z_ref[...] = z
    out_ref[...] = _segmax(z, bcol_ref[...], G)


def _head1_body(x1_ref, lW_ref, lb_ref, bcol_ref, z0_ref, out0_ref,
                out_ref, Z_ref):
    G = out_ref.shape[0]
    z = jnp.dot(x1_ref[...], lW_ref[...], preferred_element_type=jnp.float32)
    z = z + lb_ref[...]
    Z_ref[...] = z0_ref[...] + z
    out_ref[...] = out0_ref[...] + _segmax(z, bcol_ref[...], G)


def _sc_agg(x0, src, dst, zeros):
    N, D = x0.shape
    E = src.shape[0]
    NW = 32
    EPW = E // NW
    K = 80
    NCH = EPW // K
    RPT = N // 16

    mesh = plsc.VectorSubcoreMesh(core_axis_name="c", subcore_axis_name="s")

    @functools.partial(
        pl.kernel,
        out_type=jax.ShapeDtypeStruct((2 * N, D), jnp.float32),
        mesh=mesh,
        scratch_types=[
            pltpu.VMEM((K,), jnp.int32),
            pltpu.VMEM((K,), jnp.int32),
            pltpu.VMEM((K, D), jnp.float32),
            pltpu.VMEM_SHARED((N, D), jnp.float32),
            pltpu.SemaphoreType.DMA,
        ],
    )
    def k(x0_hbm, src_hbm, dst_hbm, zero_hbm, out_hbm,
          sidx, didx, rows, aggsp, sem):
        c = lax.axis_index("c")
        s = lax.axis_index("s")
        wid = s * 2 + c
        pltpu.sync_copy(zero_hbm.at[pl.ds(s * RPT, RPT)],
                        aggsp.at[pl.ds(s * RPT, RPT)])
        plsc.subcore_barrier()
        base = wid * EPW

        def body(i, carry):
            off = base + i * K
            pltpu.sync_copy(src_hbm.at[pl.ds(off, K)], sidx)
            pltpu.sync_copy(dst_hbm.at[pl.ds(off, K)], didx)
            pltpu.async_copy(x0_hbm.at[sidx], rows, sem).wait()
            pltpu.sync_copy(rows, aggsp.at[didx], add=True)
            return carry

        lax.fori_loop(0, NCH, body, 0)
        plsc.subcore_barrier()
        pltpu.sync_copy(aggsp.at[pl.ds(s * RPT, RPT)],
                        out_hbm.at[pl.ds(c * N + s * RPT, RPT)])

    return k(x0, src, dst, zeros)


def kernel(x, fW1, fb1, fg1, fbe1, fW2, fb2, fg2, fbe2, l0W, l0b,
           cW1, cb1, cg1, cbe1, cW2, cb2, cg2, cbe2, l1W, l1b,
           edge_index, batch):
    N, D = x.shape
    H = fW1.shape[1]
    T = l0W.shape[1]
    G = 64

    src = edge_index[0]
    dst = edge_index[1]
    bcol = batch.reshape(N, 1)
    r1 = lambda v: v.reshape(1, -1)

    x0 = pl.pallas_call(
        _mlp_body,
        out_shape=jax.ShapeDtypeStruct((N, H), jnp.float32),
    )(x, fW1, r1(fb1), r1(fg1), r1(fbe1), fW2, r1(fb2), r1(fg2), r1(fbe2))

    zeros = jnp.zeros((N, H), jnp.float32)
    aggp = _sc_agg(x0, src, dst, zeros)

    z0, out0 = pl.pallas_call(
        _head0_body,
        out_shape=(
            jax.ShapeDtypeStruct((N, T), jnp.float32),
            jax.ShapeDtypeStruct((G, T), jnp.float32),
        ),
    )(x0, l0W, r1(l0b), bcol)

    x1 = pl.pallas_call(
        _mlp3_body,
        out_shape=jax.ShapeDtypeStruct((N, H), jnp.float32),
    )(x0, aggp[:N], aggp[N:], cW1, r1(cb1), r1(cg1), r1(cbe1),
      cW2, r1(cb2), r1(cg2), r1(cbe2))

    out, Z = pl.pallas_call(
        _head1_body,
        out_shape=(
            jax.ShapeDtypeStruct((G, T), jnp.float32),
            jax.ShapeDtypeStruct((N, T), jnp.float32),
        ),
    )(x1, l1W, r1(l1b), bcol, z0, out0)

    return (out, Z, x1)


# speedup vs baseline: 3.9987x; 3.9987x over previous
import functools

import jax
import jax.numpy as jnp
from jax import lax
from jax.experimental import pallas as pl
from jax.experimental.pallas import tpu as pltpu
from jax.experimental.pallas import tpu_sc as plsc

NEG = -1e30


def _bn_relu(h, g, be):
    m = jnp.mean(h, axis=0, keepdims=True)
    v = jnp.mean((h - m) ** 2, axis=0, keepdims=True)
    return jnp.maximum(g * (h - m) * lax.rsqrt(v + 1e-5) + be, 0.0)


def _mlp_body(x_ref, W1_ref, b1_ref, g1_ref, be1_ref, W2_ref, b2_ref,
              g2_ref, be2_ref, o_ref):
    h = jnp.dot(x_ref[...], W1_ref[...], preferred_element_type=jnp.float32)
    a = _bn_relu(h + b1_ref[...], g1_ref[...], be1_ref[...])
    h2 = jnp.dot(a, W2_ref[...], preferred_element_type=jnp.float32)
    o_ref[...] = _bn_relu(h2 + b2_ref[...], g2_ref[...], be2_ref[...])


def _mlp3_body(x_ref, p0_ref, p1_ref, W1_ref, b1_ref, g1_ref, be1_ref,
               W2_ref, b2_ref, g2_ref, be2_ref, o_ref):
    y = x_ref[...] + p0_ref[...] + p1_ref[...]
    h = jnp.dot(y, W1_ref[...], preferred_element_type=jnp.float32)
    a = _bn_relu(h + b1_ref[...], g1_ref[...], be1_ref[...])
    h2 = jnp.dot(a, W2_ref[...], preferred_element_type=jnp.float32)
    o_ref[...] = _bn_relu(h2 + b2_ref[...], g2_ref[...], be2_ref[...])


def _segmax(z, bcol, G):
    rows = []
    for g in range(G):
        rows.append(jnp.max(jnp.where(bcol == g, z, NEG), axis=0, keepdims=True))
    return jnp.concatenate(rows, axis=0)


def _head0_body(x0_ref, lW_ref, lb_ref, bcol_ref, z_ref, out_ref):
    G = out_ref.shape[0]
    z = jnp.dot(x0_ref[...], lW_ref[...], preferred_element_type=jnp.float32)
    z = z + lb_ref[...]
    z_ref[...] = z
    out_ref[...] = _segmax(z, bcol_ref[...], G)


def _head1_body(x1_ref, lW_ref, lb_ref, bcol_ref, z0_ref, out0_ref,
                out_ref, Z_ref):
    G = out_ref.shape[0]
    z = jnp.dot(x1_ref[...], lW_ref[...], preferred_element_type=jnp.float32)
    z = z + lb_ref[...]
    Z_ref[...] = z0_ref[...] + z
    out_ref[...] = out0_ref[...] + _segmax(z, bcol_ref[...], G)


def _sc_agg(x0, src, dst, zeros):
    N, D = x0.shape
    E = src.shape[0]
    NW = 32
    EPW = E // NW
    K = 80
    NCH = EPW // K
    RPT = (N // 16) & ~7
    TAIL = N - 16 * RPT

    mesh = plsc.VectorSubcoreMesh(core_axis_name="c", subcore_axis_name="s")

    @functools.partial(
        pl.kernel,
        out_type=jax.ShapeDtypeStruct((2 * N, D), jnp.float32),
        mesh=mesh,
        scratch_types=[
            pltpu.VMEM((K,), jnp.int32),
            pltpu.VMEM((K,), jnp.int32),
            pltpu.VMEM((K, D), jnp.float32),
            pltpu.VMEM_SHARED((N, D), jnp.float32),
            pltpu.SemaphoreType.DMA,
        ],
    )
    def k(x0_hbm, src_hbm, dst_hbm, zero_hbm, out_hbm,
          sidx, didx, rows, aggsp, sem):
        c = lax.axis_index("c")
        s = lax.axis_index("s")
        wid = s * 2 + c
        pltpu.sync_copy(zero_hbm.at[pl.ds(s * RPT, RPT)],
                        aggsp.at[pl.ds(s * RPT, RPT)])

        @pl.when(s == 15)
        def _():
            pltpu.sync_copy(zero_hbm.at[pl.ds(16 * RPT, TAIL)],
                            aggsp.at[pl.ds(16 * RPT, TAIL)])

        plsc.subcore_barrier()
        base = wid * EPW

        def body(i, carry):
            off = base + i * K
            pltpu.sync_copy(src_hbm.at[pl.ds(off, K)], sidx)
            pltpu.sync_copy(dst_hbm.at[pl.ds(off, K)], didx)
            pltpu.async_copy(x0_hbm.at[sidx], rows, sem).wait()
            pltpu.sync_copy(rows, aggsp.at[didx], add=True)
            return carry

        lax.fori_loop(0, NCH, body, 0)
        plsc.subcore_barrier()
        pltpu.sync_copy(aggsp.at[pl.ds(s * RPT, RPT)],
                        out_hbm.at[pl.ds(c * N + s * RPT, RPT)])

        @pl.when(s == 15)
        def _():
            pltpu.sync_copy(aggsp.at[pl.ds(16 * RPT, TAIL)],
                            out_hbm.at[pl.ds(c * N + 16 * RPT, TAIL)])

    return k(x0, src, dst, zeros)


def kernel(x, fW1, fb1, fg1, fbe1, fW2, fb2, fg2, fbe2, l0W, l0b,
           cW1, cb1, cg1, cbe1, cW2, cb2, cg2, cbe2, l1W, l1b,
           edge_index, batch):
    N, D = x.shape
    H = fW1.shape[1]
    T = l0W.shape[1]
    G = 64

    src = edge_index[0]
    dst = edge_index[1]
    bcol = batch.reshape(N, 1)
    r1 = lambda v: v.reshape(1, -1)

    x0 = pl.pallas_call(
        _mlp_body,
        out_shape=jax.ShapeDtypeStruct((N, H), jnp.float32),
    )(x, fW1, r1(fb1), r1(fg1), r1(fbe1), fW2, r1(fb2), r1(fg2), r1(fbe2))

    zeros = jnp.zeros((N, H), jnp.float32)
    aggp = _sc_agg(x0, src, dst, zeros)

    z0, out0 = pl.pallas_call(
        _head0_body,
        out_shape=(
            jax.ShapeDtypeStruct((N, T), jnp.float32),
            jax.ShapeDtypeStruct((G, T), jnp.float32),
        ),
    )(x0, l0W, r1(l0b), bcol)

    x1 = pl.pallas_call(
        _mlp3_body,
        out_shape=jax.ShapeDtypeStruct((N, H), jnp.float32),
    )(x0, aggp[:N], aggp[N:], cW1, r1(cb1), r1(cg1), r1(cbe1),
      cW2, r1(cb2), r1(cg2), r1(cbe2))

    out, Z = pl.pallas_call(
        _head1_body,
        out_shape=(
            jax.ShapeDtypeStruct((G, T), jnp.float32),
            jax.ShapeDtypeStruct((N, T), jnp.float32),
        ),
    )(x1, l1W, r1(l1b), bcol, z0, out0)

    return (out, Z, x1)


# speedup vs baseline: 6.3057x; 1.5769x over previous
import functools

import jax
import jax.numpy as jnp
from jax import lax
from jax.experimental import pallas as pl
from jax.experimental.pallas import tpu as pltpu
from jax.experimental.pallas import tpu_sc as plsc

NEG = -1e30


def _bn_relu(h, g, be):
    m = jnp.mean(h, axis=0, keepdims=True)
    v = jnp.mean((h - m) ** 2, axis=0, keepdims=True)
    return jnp.maximum(g * (h - m) * lax.rsqrt(v + 1e-5) + be, 0.0)


def _mlp_body(x_ref, W1_ref, b1_ref, g1_ref, be1_ref, W2_ref, b2_ref,
              g2_ref, be2_ref, o_ref):
    h = jnp.dot(x_ref[...], W1_ref[...], preferred_element_type=jnp.float32)
    a = _bn_relu(h + b1_ref[...], g1_ref[...], be1_ref[...])
    h2 = jnp.dot(a, W2_ref[...], preferred_element_type=jnp.float32)
    o_ref[...] = _bn_relu(h2 + b2_ref[...], g2_ref[...], be2_ref[...])


def _mlp3_body(x_ref, p0_ref, p1_ref, W1_ref, b1_ref, g1_ref, be1_ref,
               W2_ref, b2_ref, g2_ref, be2_ref, o_ref):
    y = x_ref[...] + p0_ref[...] + p1_ref[...]
    h = jnp.dot(y, W1_ref[...], preferred_element_type=jnp.float32)
    a = _bn_relu(h + b1_ref[...], g1_ref[...], be1_ref[...])
    h2 = jnp.dot(a, W2_ref[...], preferred_element_type=jnp.float32)
    o_ref[...] = _bn_relu(h2 + b2_ref[...], g2_ref[...], be2_ref[...])


def _segmax(z, bcol, G):
    rows = []
    for g in range(G):
        rows.append(jnp.max(jnp.where(bcol == g, z, NEG), axis=0, keepdims=True))
    return jnp.concatenate(rows, axis=0)


def _head0_body(x0_ref, lW_ref, lb_ref, bcol_ref, z_ref, out_ref):
    G = out_ref.shape[0]
    z = jnp.dot(x0_ref[...], lW_ref[...], preferred_element_type=jnp.float32)
    z = z + lb_ref[...]
    z_ref[...] = z
    out_ref[...] = _segmax(z, bcol_ref[...], G)


def _head1_body(x1_ref, lW_ref, lb_ref, bcol_ref, z0_ref, out0_ref,
                out_ref, Z_ref):
    G = out_ref.shape[0]
    z = jnp.dot(x1_ref[...], lW_ref[...], preferred_element_type=jnp.float32)
    z = z + lb_ref[...]
    Z_ref[...] = z0_ref[...] + z
    out_ref[...] = out0_ref[...] + _segmax(z, bcol_ref[...], G)


def _sc_agg(x0, src, dst, zeros):
    N, D = x0.shape
    E = src.shape[0]
    NW = 32
    EPW = E // NW
    K = 80
    NCH = EPW // K
    RPT = (N // 16) & ~7
    TAIL = N - 16 * RPT

    NG = 5
    GR = NCH // NG
    src4 = src.reshape(NW, NG, GR, K)
    dst4 = dst.reshape(NW, NG, GR, K)

    mesh = plsc.VectorSubcoreMesh(core_axis_name="c", subcore_axis_name="s")

    @functools.partial(
        pl.kernel,
        out_type=jax.ShapeDtypeStruct((2 * N, D), jnp.float32),
        mesh=mesh,
        scratch_types=[
            pltpu.VMEM((GR, K), jnp.int32),
            pltpu.VMEM((GR, K), jnp.int32),
            pltpu.VMEM((K, D), jnp.float32),
            pltpu.VMEM((K, D), jnp.float32),
            pltpu.VMEM_SHARED((N, D), jnp.float32),
            pltpu.SemaphoreType.DMA,
            pltpu.SemaphoreType.DMA,
        ],
    )
    def k(x0_hbm, src_hbm, dst_hbm, zero_hbm, out_hbm,
          sidx, didx, rows0, rows1, aggsp, sem0, sem1):
        c = lax.axis_index("c")
        s = lax.axis_index("s")
        wid = s * 2 + c
        pltpu.sync_copy(zero_hbm.at[pl.ds(s * RPT, RPT)],
                        aggsp.at[pl.ds(s * RPT, RPT)])

        @pl.when(s == 15)
        def _():
            pltpu.sync_copy(zero_hbm.at[pl.ds(16 * RPT, TAIL)],
                            aggsp.at[pl.ds(16 * RPT, TAIL)])

        plsc.subcore_barrier()

        def group(g, carry):
            pltpu.sync_copy(src_hbm.at[wid, g], sidx)
            pltpu.sync_copy(dst_hbm.at[wid, g], didx)
            pltpu.async_copy(x0_hbm.at[sidx.at[0]], rows0, sem0)

            def body(t, carry2):
                a = 2 * t + 1
                pltpu.async_copy(x0_hbm.at[sidx.at[a]], rows1, sem1)
                pltpu.make_async_copy(x0_hbm.at[sidx.at[a]], rows0, sem0).wait()
                pltpu.sync_copy(rows0, aggsp.at[didx.at[2 * t]], add=True)
                pltpu.async_copy(x0_hbm.at[sidx.at[a + 1]], rows0, sem0)
                pltpu.make_async_copy(x0_hbm.at[sidx.at[a]], rows1, sem1).wait()
                pltpu.sync_copy(rows1, aggsp.at[didx.at[a]], add=True)
                return carry2

            lax.fori_loop(0, (GR - 1) // 2, body, 0)
            pltpu.make_async_copy(x0_hbm.at[sidx.at[0]], rows0, sem0).wait()
            pltpu.sync_copy(rows0, aggsp.at[didx.at[GR - 1]], add=True)
            return carry

        lax.fori_loop(0, NG, group, 0)

        plsc.subcore_barrier()
        pltpu.sync_copy(aggsp.at[pl.ds(s * RPT, RPT)],
                        out_hbm.at[pl.ds(c * N + s * RPT, RPT)])

        @pl.when(s == 15)
        def _():
            pltpu.sync_copy(aggsp.at[pl.ds(16 * RPT, TAIL)],
                            out_hbm.at[pl.ds(c * N + 16 * RPT, TAIL)])

    return k(x0, src4, dst4, zeros)


def kernel(x, fW1, fb1, fg1, fbe1, fW2, fb2, fg2, fbe2, l0W, l0b,
           cW1, cb1, cg1, cbe1, cW2, cb2, cg2, cbe2, l1W, l1b,
           edge_index, batch):
    N, D = x.shape
    H = fW1.shape[1]
    T = l0W.shape[1]
    G = 64

    src = edge_index[0]
    dst = edge_index[1]
    bcol = batch.reshape(N, 1)
    r1 = lambda v: v.reshape(1, -1)

    x0 = pl.pallas_call(
        _mlp_body,
        out_shape=jax.ShapeDtypeStruct((N, H), jnp.float32),
    )(x, fW1, r1(fb1), r1(fg1), r1(fbe1), fW2, r1(fb2), r1(fg2), r1(fbe2))

    zeros = jnp.zeros((N, H), jnp.float32)
    aggp = _sc_agg(x0, src, dst, zeros)

    z0, out0 = pl.pallas_call(
        _head0_body,
        out_shape=(
            jax.ShapeDtypeStruct((N, T), jnp.float32),
            jax.ShapeDtypeStruct((G, T), jnp.float32),
        ),
    )(x0, l0W, r1(l0b), bcol)

    x1 = pl.pallas_call(
        _mlp3_body,
        out_shape=jax.ShapeDtypeStruct((N, H), jnp.float32),
    )(x0, aggp[:N], aggp[N:], cW1, r1(cb1), r1(cg1), r1(cbe1),
      cW2, r1(cb2), r1(cg2), r1(cbe2))

    out, Z = pl.pallas_call(
        _head1_body,
        out_shape=(
            jax.ShapeDtypeStruct((G, T), jnp.float32),
            jax.ShapeDtypeStruct((N, T), jnp.float32),
        ),
    )(x1, l1W, r1(l1b), bcol, z0, out0)

    return (out, Z, x1)


# speedup vs baseline: 8.7054x; 1.3806x over previous
import functools

import jax
import jax.numpy as jnp
from jax import lax
from jax.experimental import pallas as pl
from jax.experimental.pallas import tpu as pltpu
from jax.experimental.pallas import tpu_sc as plsc

NEG = -1e30


def _bn_relu(h, g, be):
    m = jnp.mean(h, axis=0, keepdims=True)
    v = jnp.mean((h - m) ** 2, axis=0, keepdims=True)
    return jnp.maximum(g * (h - m) * lax.rsqrt(v + 1e-5) + be, 0.0)


def _mlp_body(x_ref, W1_ref, b1_ref, g1_ref, be1_ref, W2_ref, b2_ref,
              g2_ref, be2_ref, o_ref):
    h = jnp.dot(x_ref[...], W1_ref[...], preferred_element_type=jnp.float32)
    a = _bn_relu(h + b1_ref[...], g1_ref[...], be1_ref[...])
    h2 = jnp.dot(a, W2_ref[...], preferred_element_type=jnp.float32)
    o_ref[...] = _bn_relu(h2 + b2_ref[...], g2_ref[...], be2_ref[...])


def _mlp3_body(x_ref, p0_ref, p1_ref, W1_ref, b1_ref, g1_ref, be1_ref,
               W2_ref, b2_ref, g2_ref, be2_ref, o_ref):
    y = x_ref[...] + p0_ref[...] + p1_ref[...]
    h = jnp.dot(y, W1_ref[...], preferred_element_type=jnp.float32)
    a = _bn_relu(h + b1_ref[...], g1_ref[...], be1_ref[...])
    h2 = jnp.dot(a, W2_ref[...], preferred_element_type=jnp.float32)
    o_ref[...] = _bn_relu(h2 + b2_ref[...], g2_ref[...], be2_ref[...])


def _segmax_t(zT, bT, G):
    cols = []
    for g in range(G):
        cols.append(jnp.max(jnp.where(bT == g, zT, NEG), axis=1, keepdims=True))
    return jnp.concatenate(cols, axis=1).T


def _head0_body(x0_ref, lW_ref, lb_ref, bT_ref, z_ref, out_ref):
    G = out_ref.shape[0]
    z = jnp.dot(x0_ref[...], lW_ref[...], preferred_element_type=jnp.float32)
    z = z + lb_ref[...]
    z_ref[...] = z
    out_ref[...] = _segmax_t(z.T, bT_ref[...], G)


def _head1_body(x1_ref, lW_ref, lb_ref, bT_ref, z0_ref, out0_ref,
                out_ref, Z_ref):
    G = out_ref.shape[0]
    z = jnp.dot(x1_ref[...], lW_ref[...], preferred_element_type=jnp.float32)
    z = z + lb_ref[...]
    Z_ref[...] = z0_ref[...] + z
    out_ref[...] = out0_ref[...] + _segmax_t(z.T, bT_ref[...], G)


def _sc_agg(x0, src, dst, zeros):
    N, D = x0.shape
    E = src.shape[0]
    NW = 32
    EPW = E // NW
    K = 80
    NCH = EPW // K
    RPT = (N // 16) & ~7
    TAIL = N - 16 * RPT

    NG = 5
    GR = NCH // NG
    src4 = src.reshape(NW, NG, GR, K)
    dst4 = dst.reshape(NW, NG, GR, K)

    mesh = plsc.VectorSubcoreMesh(core_axis_name="c", subcore_axis_name="s")

    @functools.partial(
        pl.kernel,
        out_type=jax.ShapeDtypeStruct((2 * N, D), jnp.float32),
        mesh=mesh,
        scratch_types=[
            pltpu.VMEM((GR, K), jnp.int32),
            pltpu.VMEM((GR, K), jnp.int32),
            pltpu.VMEM((K, D), jnp.float32),
            pltpu.VMEM((K, D), jnp.float32),
            pltpu.VMEM_SHARED((N, D), jnp.float32),
            pltpu.SemaphoreType.DMA,
            pltpu.SemaphoreType.DMA,
        ],
    )
    def k(x0_hbm, src_hbm, dst_hbm, zero_hbm, out_hbm,
          sidx, didx, rows0, rows1, aggsp, sem0, sem1):
        c = lax.axis_index("c")
        s = lax.axis_index("s")
        wid = s * 2 + c
        pltpu.sync_copy(zero_hbm.at[pl.ds(s * RPT, RPT)],
                        aggsp.at[pl.ds(s * RPT, RPT)])

        @pl.when(s == 15)
        def _():
            pltpu.sync_copy(zero_hbm.at[pl.ds(16 * RPT, TAIL)],
                            aggsp.at[pl.ds(16 * RPT, TAIL)])

        plsc.subcore_barrier()

        def group(g, carry):
            pltpu.sync_copy(src_hbm.at[wid, g], sidx)
            pltpu.sync_copy(dst_hbm.at[wid, g], didx)
            pltpu.async_copy(x0_hbm.at[sidx.at[0]], rows0, sem0)

            def body(t, carry2):
                a = 2 * t + 1
                pltpu.async_copy(x0_hbm.at[sidx.at[a]], rows1, sem1)
                pltpu.make_async_copy(x0_hbm.at[sidx.at[a]], rows0, sem0).wait()
                pltpu.sync_copy(rows0, aggsp.at[didx.at[2 * t]], add=True)
                pltpu.async_copy(x0_hbm.at[sidx.at[a + 1]], rows0, sem0)
                pltpu.make_async_copy(x0_hbm.at[sidx.at[a]], rows1, sem1).wait()
                pltpu.sync_copy(rows1, aggsp.at[didx.at[a]], add=True)
                return carry2

            lax.fori_loop(0, (GR - 1) // 2, body, 0)
            pltpu.make_async_copy(x0_hbm.at[sidx.at[0]], rows0, sem0).wait()
            pltpu.sync_copy(rows0, aggsp.at[didx.at[GR - 1]], add=True)
            return carry

        lax.fori_loop(0, NG, group, 0)

        plsc.subcore_barrier()
        pltpu.sync_copy(aggsp.at[pl.ds(s * RPT, RPT)],
                        out_hbm.at[pl.ds(c * N + s * RPT, RPT)])

        @pl.when(s == 15)
        def _():
            pltpu.sync_copy(aggsp.at[pl.ds(16 * RPT, TAIL)],
                            out_hbm.at[pl.ds(c * N + 16 * RPT, TAIL)])

    return k(x0, src4, dst4, zeros)


def kernel(x, fW1, fb1, fg1, fbe1, fW2, fb2, fg2, fbe2, l0W, l0b,
           cW1, cb1, cg1, cbe1, cW2, cb2, cg2, cbe2, l1W, l1b,
           edge_index, batch):
    N, D = x.shape
    H = fW1.shape[1]
    T = l0W.shape[1]
    G = 64

    src = edge_index[0]
    dst = edge_index[1]
    bT = batch.reshape(1, N)
    r1 = lambda v: v.reshape(1, -1)

    x0 = pl.pallas_call(
        _mlp_body,
        out_shape=jax.ShapeDtypeStruct((N, H), jnp.float32),
    )(x, fW1, r1(fb1), r1(fg1), r1(fbe1), fW2, r1(fb2), r1(fg2), r1(fbe2))

    zeros = jnp.zeros((N, H), jnp.float32)
    aggp = _sc_agg(x0, src, dst, zeros)

    z0, out0 = pl.pallas_call(
        _head0_body,
        out_shape=(
            jax.ShapeDtypeStruct((N, T), jnp.float32),
            jax.ShapeDtypeStruct((G, T), jnp.float32),
        ),
    )(x0, l0W, r1(l0b), bT)

    x1 = pl.pallas_call(
        _mlp3_body,
        out_shape=jax.ShapeDtypeStruct((N, H), jnp.float32),
    )(x0, aggp[:N], aggp[N:], cW1, r1(cb1), r1(cg1), r1(cbe1),
      cW2, r1(cb2), r1(cg2), r1(cbe2))

    out, Z = pl.pallas_call(
        _head1_body,
        out_shape=(
            jax.ShapeDtypeStruct((G, T), jnp.float32),
            jax.ShapeDtypeStruct((N, T), jnp.float32),
        ),
    )(x1, l1W, r1(l1b), bT, z0, out0)

    return (out, Z, x1)


# speedup vs baseline: 9.2017x; 1.0570x over previous
import functools

import jax
import jax.numpy as jnp
from jax import lax
from jax.experimental import pallas as pl
from jax.experimental.pallas import tpu as pltpu
from jax.experimental.pallas import tpu_sc as plsc

NEG = -1e30


def _bn_relu(h, g, be):
    m = jnp.mean(h, axis=0, keepdims=True)
    v = jnp.mean((h - m) ** 2, axis=0, keepdims=True)
    return jnp.maximum(g * (h - m) * lax.rsqrt(v + 1e-5) + be, 0.0)


def _mlp(x, W1, b1, g1, be1, W2, b2, g2, be2):
    h = jnp.dot(x, W1, preferred_element_type=jnp.float32)
    a = _bn_relu(h + b1, g1, be1)
    h2 = jnp.dot(a, W2, preferred_element_type=jnp.float32)
    return _bn_relu(h2 + b2, g2, be2)


def _segmax_t(zT, bT, G):
    cols = []
    for g in range(G):
        cols.append(jnp.max(jnp.where(bT == g, zT, NEG), axis=1, keepdims=True))
    return jnp.concatenate(cols, axis=1).T


def _layer0_body(x_ref, W1_ref, b1_ref, g1_ref, be1_ref, W2_ref, b2_ref,
                 g2_ref, be2_ref, lW_ref, lb_ref, bT_ref,
                 x0_ref, z0_ref, out0_ref):
    G = out0_ref.shape[0]
    x0 = _mlp(x_ref[...], W1_ref[...], b1_ref[...], g1_ref[...], be1_ref[...],
              W2_ref[...], b2_ref[...], g2_ref[...], be2_ref[...])
    x0_ref[...] = x0
    z = jnp.dot(x0, lW_ref[...], preferred_element_type=jnp.float32)
    z = z + lb_ref[...]
    z0_ref[...] = z
    out0_ref[...] = _segmax_t(z.T, bT_ref[...], G)


def _layer1_body(x0_ref, p0_ref, p1_ref, W1_ref, b1_ref, g1_ref, be1_ref,
                 W2_ref, b2_ref, g2_ref, be2_ref, lW_ref, lb_ref, bT_ref,
                 z0_ref, out0_ref, out_ref, Z_ref, x1_ref):
    G = out_ref.shape[0]
    y = x0_ref[...] + p0_ref[...] + p1_ref[...]
    x1 = _mlp(y, W1_ref[...], b1_ref[...], g1_ref[...], be1_ref[...],
              W2_ref[...], b2_ref[...], g2_ref[...], be2_ref[...])
    x1_ref[...] = x1
    z = jnp.dot(x1, lW_ref[...], preferred_element_type=jnp.float32)
    z = z + lb_ref[...]
    Z_ref[...] = z0_ref[...] + z
    out_ref[...] = out0_ref[...] + _segmax_t(z.T, bT_ref[...], G)


def _sc_agg(x0, src, dst, zeros):
    N, D = x0.shape
    E = src.shape[0]
    NW = 32
    EPW = E // NW
    K = 80
    NCH = EPW // K
    NG = 5
    GR = NCH // NG
    RPT = (N // 16) & ~7
    TAIL = N - 16 * RPT

    src4 = src.reshape(NW, NG, GR, K)
    dst4 = dst.reshape(NW, NG, GR, K)

    mesh = plsc.VectorSubcoreMesh(core_axis_name="c", subcore_axis_name="s")

    @functools.partial(
        pl.kernel,
        out_type=jax.ShapeDtypeStruct((2 * N, D), jnp.float32),
        mesh=mesh,
        scratch_types=[
            pltpu.VMEM((GR, K), jnp.int32),
            pltpu.VMEM((GR, K), jnp.int32),
            pltpu.VMEM((K, D), jnp.float32),
            pltpu.VMEM((K, D), jnp.float32),
            pltpu.VMEM((K, D), jnp.float32),
            pltpu.VMEM_SHARED((N, D), jnp.float32),
            pltpu.SemaphoreType.DMA,
            pltpu.SemaphoreType.DMA,
            pltpu.SemaphoreType.DMA,
            pltpu.SemaphoreType.DMA,
            pltpu.SemaphoreType.DMA,
            pltpu.SemaphoreType.DMA,
        ],
    )
    def k(x0_hbm, src_hbm, dst_hbm, zero_hbm, out_hbm,
          sidx, didx, rows0, rows1, rows2, aggsp,
          gs0, gs1, gs2, ss0, ss1, ss2):
        rows = (rows0, rows1, rows2)
        gs = (gs0, gs1, gs2)
        ss = (ss0, ss1, ss2)
        c = lax.axis_index("c")
        s = lax.axis_index("s")
        wid = s * 2 + c
        pltpu.sync_copy(zero_hbm.at[pl.ds(s * RPT, RPT)],
                        aggsp.at[pl.ds(s * RPT, RPT)])

        @pl.when(s == 15)
        def _():
            pltpu.sync_copy(zero_hbm.at[pl.ds(16 * RPT, TAIL)],
                            aggsp.at[pl.ds(16 * RPT, TAIL)])

        plsc.subcore_barrier()

        def group(g, carry):
            pltpu.sync_copy(src_hbm.at[wid, g], sidx)
            pltpu.sync_copy(dst_hbm.at[wid, g], didx)
            gd = [None] * GR
            sd = [None] * GR
            gd[0] = pltpu.async_copy(x0_hbm.at[sidx.at[0]], rows[0], gs[0])
            gd[1] = pltpu.async_copy(x0_hbm.at[sidx.at[1]], rows[1], gs[1])
            for i in range(GR):
                u = i % 3
                gd[i].wait()
                sd[i] = pltpu.async_copy(rows[u], aggsp.at[didx.at[i]],
                                         ss[u], add=True)
                if i + 2 < GR:
                    v = (i + 2) % 3
                    if i >= 1:
                        sd[i - 1].wait()
                    gd[i + 2] = pltpu.async_copy(x0_hbm.at[sidx.at[i + 2]],
                                                 rows[v], gs[v])
            sd[GR - 3].wait()
            sd[GR - 2].wait()
            sd[GR - 1].wait()
            return carry

        lax.fori_loop(0, NG, group, 0)

        plsc.subcore_barrier()
        pltpu.sync_copy(aggsp.at[pl.ds(s * RPT, RPT)],
                        out_hbm.at[pl.ds(c * N + s * RPT, RPT)])

        @pl.when(s == 15)
        def _():
            pltpu.sync_copy(aggsp.at[pl.ds(16 * RPT, TAIL)],
                            out_hbm.at[pl.ds(c * N + 16 * RPT, TAIL)])

    return k(x0, src4, dst4, zeros)


def kernel(x, fW1, fb1, fg1, fbe1, fW2, fb2, fg2, fbe2, l0W, l0b,
           cW1, cb1, cg1, cbe1, cW2, cb2, cg2, cbe2, l1W, l1b,
           edge_index, batch):
    N, D = x.shape
    H = fW1.shape[1]
    T = l0W.shape[1]
    G = 64

    src = edge_index[0]
    dst = edge_index[1]
    bT = batch.reshape(1, N)
    r1 = lambda v: v.reshape(1, -1)

    x0, z0, out0 = pl.pallas_call(
        _layer0_body,
        out_shape=(
            jax.ShapeDtypeStruct((N, H), jnp.float32),
            jax.ShapeDtypeStruct((N, T), jnp.float32),
            jax.ShapeDtypeStruct((G, T), jnp.float32),
        ),
    )(x, fW1, r1(fb1), r1(fg1), r1(fbe1), fW2, r1(fb2), r1(fg2), r1(fbe2),
      l0W, r1(l0b), bT)

    zeros = jnp.zeros((N, H), jnp.float32)
    aggp = _sc_agg(x0, src, dst, zeros)

    out, Z, x1 = pl.pallas_call(
        _layer1_body,
        out_shape=(
            jax.ShapeDtypeStruct((G, T), jnp.float32),
            jax.ShapeDtypeStruct((N, T), jnp.float32),
            jax.ShapeDtypeStruct((N, H), jnp.float32),
        ),
    )(x0, aggp[:N], aggp[N:], cW1, r1(cb1), r1(cg1), r1(cbe1),
      cW2, r1(cb2), r1(cg2), r1(cbe2), l1W, r1(l1b), bT, z0, out0)

    return (out, Z, x1)


# speedup vs baseline: 10.8811x; 1.1825x over previous
import functools

import jax
import jax.numpy as jnp
from jax import lax
from jax.experimental import pallas as pl
from jax.experimental.pallas import tpu as pltpu
from jax.experimental.pallas import tpu_sc as plsc

NEG = -1e30


def _bn_relu(h, g, be):
    m = jnp.mean(h, axis=0, keepdims=True)
    v = jnp.mean((h - m) ** 2, axis=0, keepdims=True)
    return jnp.maximum(g * (h - m) * lax.rsqrt(v + 1e-5) + be, 0.0)


def _mlp(x, W1, b1, g1, be1, W2, b2, g2, be2):
    h = jnp.dot(x, W1, preferred_element_type=jnp.float32)
    a = _bn_relu(h + b1, g1, be1)
    h2 = jnp.dot(a, W2, preferred_element_type=jnp.float32)
    return _bn_relu(h2 + b2, g2, be2)


def _segmax_t(zT, bT, G):
    cols = []
    for g in range(G):
        cols.append(jnp.max(jnp.where(bT == g, zT, NEG), axis=1, keepdims=True))
    return jnp.concatenate(cols, axis=1).T


def _mlp_body(x_ref, W1_ref, b1_ref, g1_ref, be1_ref, W2_ref, b2_ref,
              g2_ref, be2_ref, x0_ref):
    x0_ref[...] = _mlp(x_ref[...], W1_ref[...], b1_ref[...], g1_ref[...],
                       be1_ref[...], W2_ref[...], b2_ref[...], g2_ref[...],
                       be2_ref[...])


def _head0_body(x0_ref, lW_ref, lb_ref, bT_ref, z0_ref, out0_ref):
    G = out0_ref.shape[0]
    z = jnp.dot(x0_ref[...], lW_ref[...], preferred_element_type=jnp.float32)
    z = z + lb_ref[...]
    z0_ref[...] = z
    out0_ref[...] = _segmax_t(z.T, bT_ref[...], G)


def _layer1_body(x0_ref, pp_ref, W1_ref, b1_ref, g1_ref, be1_ref,
                 W2_ref, b2_ref, g2_ref, be2_ref, lW_ref, lb_ref, bT_ref,
                 z0_ref, out0_ref, out_ref, Z_ref, x1_ref):
    G = out_ref.shape[0]
    N = x0_ref.shape[0]
    y = x0_ref[...] + pp_ref[pl.ds(0, N), :] + pp_ref[pl.ds(N, N), :]
    x1 = _mlp(y, W1_ref[...], b1_ref[...], g1_ref[...], be1_ref[...],
              W2_ref[...], b2_ref[...], g2_ref[...], be2_ref[...])
    x1_ref[...] = x1
    z = jnp.dot(x1, lW_ref[...], preferred_element_type=jnp.float32)
    z = z + lb_ref[...]
    Z_ref[...] = z0_ref[...] + z
    out_ref[...] = out0_ref[...] + _segmax_t(z.T, bT_ref[...], G)


def _sc_agg(x0, edge_index, zeros):
    N, D = x0.shape
    E = edge_index.shape[1]
    NW = 32
    EPW = E // NW
    K = 80
    NCH = EPW // K
    NG = 5
    GR = NCH // NG
    RPT = (N // 16) & ~7
    TAIL = N - 16 * RPT

    eidx5 = edge_index.reshape(2, NW, NG, GR, K)

    mesh = plsc.VectorSubcoreMesh(core_axis_name="c", subcore_axis_name="s")

    @functools.partial(
        pl.kernel,
        out_type=jax.ShapeDtypeStruct((2 * N, D), jnp.float32),
        mesh=mesh,
        scratch_types=[
            pltpu.VMEM((GR, K), jnp.int32),
            pltpu.VMEM((GR, K), jnp.int32),
            pltpu.VMEM((K, D), jnp.float32),
            pltpu.VMEM((K, D), jnp.float32),
            pltpu.VMEM((K, D), jnp.float32),
            pltpu.VMEM_SHARED((N, D), jnp.float32),
            pltpu.SemaphoreType.DMA,
            pltpu.SemaphoreType.DMA,
            pltpu.SemaphoreType.DMA,
            pltpu.SemaphoreType.DMA,
            pltpu.SemaphoreType.DMA,
            pltpu.SemaphoreType.DMA,
        ],
    )
    def k(x0_hbm, eidx_hbm, zero_hbm, out_hbm,
          sidx, didx, rows0, rows1, rows2, aggsp,
          gs0, gs1, gs2, ss0, ss1, ss2):
        rows = (rows0, rows1, rows2)
        gs = (gs0, gs1, gs2)
        ss = (ss0, ss1, ss2)
        c = lax.axis_index("c")
        s = lax.axis_index("s")
        wid = s * 2 + c
        pltpu.sync_copy(zero_hbm.at[pl.ds(s * RPT, RPT)],
                        aggsp.at[pl.ds(s * RPT, RPT)])

        @pl.when(s == 15)
        def _():
            pltpu.sync_copy(zero_hbm.at[pl.ds(16 * RPT, TAIL)],
                            aggsp.at[pl.ds(16 * RPT, TAIL)])

        plsc.subcore_barrier()

        def group(g, carry):
            pltpu.sync_copy(eidx_hbm.at[0, wid, g], sidx)
            pltpu.sync_copy(eidx_hbm.at[1, wid, g], didx)
            gd = [None] * GR
            sd = [None] * GR
            gd[0] = pltpu.async_copy(x0_hbm.at[sidx.at[0]], rows[0], gs[0])
            gd[1] = pltpu.async_copy(x0_hbm.at[sidx.at[1]], rows[1], gs[1])
            for i in range(GR):
                u = i % 3
                gd[i].wait()
                sd[i] = pltpu.async_copy(rows[u], aggsp.at[didx.at[i]],
                                         ss[u], add=True)
                if i + 2 < GR:
                    v = (i + 2) % 3
                    if i >= 1:
                        sd[i - 1].wait()
                    gd[i + 2] = pltpu.async_copy(x0_hbm.at[sidx.at[i + 2]],
                                                 rows[v], gs[v])
            sd[GR - 3].wait()
            sd[GR - 2].wait()
            sd[GR - 1].wait()
            return carry

        lax.fori_loop(0, NG, group, 0)

        plsc.subcore_barrier()
        pltpu.sync_copy(aggsp.at[pl.ds(s * RPT, RPT)],
                        out_hbm.at[pl.ds(c * N + s * RPT, RPT)])

        @pl.when(s == 15)
        def _():
            pltpu.sync_copy(aggsp.at[pl.ds(16 * RPT, TAIL)],
                            out_hbm.at[pl.ds(c * N + 16 * RPT, TAIL)])

    return k(x0, eidx5, zeros)


def kernel(x, fW1, fb1, fg1, fbe1, fW2, fb2, fg2, fbe2, l0W, l0b,
           cW1, cb1, cg1, cbe1, cW2, cb2, cg2, cbe2, l1W, l1b,
           edge_index, batch):
    N, D = x.shape
    H = fW1.shape[1]
    T = l0W.shape[1]
    G = 64

    bT = batch.reshape(1, N)
    r1 = lambda v: v.reshape(1, -1)

    x0 = pl.pallas_call(
        _mlp_body,
        out_shape=jax.ShapeDtypeStruct((N, H), jnp.float32),
    )(x, fW1, r1(fb1), r1(fg1), r1(fbe1), fW2, r1(fb2), r1(fg2), r1(fbe2))

    zeros = jnp.zeros((N, H), jnp.float32)
    aggp = _sc_agg(x0, edge_index, zeros)

    z0, out0 = pl.pallas_call(
        _head0_body,
        out_shape=(
            jax.ShapeDtypeStruct((N, T), jnp.float32),
            jax.ShapeDtypeStruct((G, T), jnp.float32),
        ),
    )(x0, l0W, r1(l0b), bT)

    out, Z, x1 = pl.pallas_call(
        _layer1_body,
        out_shape=(
            jax.ShapeDtypeStruct((G, T), jnp.float32),
            jax.ShapeDtypeStruct((N, T), jnp.float32),
            jax.ShapeDtypeStruct((N, H), jnp.float32),
        ),
    )(x0, aggp, cW1, r1(cb1), r1(cg1), r1(cbe1),
      cW2, r1(cb2), r1(cg2), r1(cbe2), l1W, r1(l1b), bT, z0, out0)

    return (out, Z, x1)


# speedup vs baseline: 10.9303x; 1.0045x over previous
import functools

import jax
import jax.numpy as jnp
from jax import lax
from jax.experimental import pallas as pl
from jax.experimental.pallas import tpu as pltpu
from jax.experimental.pallas import tpu_sc as plsc

NEG = -1e30


def _bn_relu(h, g, be):
    m = jnp.mean(h, axis=0, keepdims=True)
    v = jnp.mean((h - m) ** 2, axis=0, keepdims=True)
    return jnp.maximum(g * (h - m) * lax.rsqrt(v + 1e-5) + be, 0.0)


def _mlp(x, W1, b1, g1, be1, W2, b2, g2, be2):
    h = jnp.dot(x, W1, preferred_element_type=jnp.float32)
    a = _bn_relu(h + b1, g1, be1)
    h2 = jnp.dot(a, W2, preferred_element_type=jnp.float32)
    return _bn_relu(h2 + b2, g2, be2)


def _segmax_t(zT, bT, G):
    cols = []
    for g in range(G):
        cols.append(jnp.max(jnp.where(bT == g, zT, NEG), axis=1, keepdims=True))
    return jnp.concatenate(cols, axis=1).T


def _mlp_body(x_ref, W1_ref, b1_ref, g1_ref, be1_ref, W2_ref, b2_ref,
              g2_ref, be2_ref, x0_ref):
    x0_ref[...] = _mlp(x_ref[...], W1_ref[...], b1_ref[...], g1_ref[...],
                       be1_ref[...], W2_ref[...], b2_ref[...], g2_ref[...],
                       be2_ref[...])


def _head0_body(x0_ref, lW_ref, lb_ref, bT_ref, z0_ref, out0_ref):
    G = out0_ref.shape[0]
    z = jnp.dot(x0_ref[...], lW_ref[...], preferred_element_type=jnp.float32)
    z = z + lb_ref[...]
    z0_ref[...] = z
    out0_ref[...] = _segmax_t(z.T, bT_ref[...], G)


def _layer1_body(pp_ref, W1_ref, b1_ref, g1_ref, be1_ref,
                 W2_ref, b2_ref, g2_ref, be2_ref, lW_ref, lb_ref, bT_ref,
                 z0_ref, out0_ref, out_ref, Z_ref, x1_ref):
    G = out_ref.shape[0]
    N = pp_ref.shape[0] // 2
    y = pp_ref[pl.ds(0, N), :] + pp_ref[pl.ds(N, N), :]
    x1 = _mlp(y, W1_ref[...], b1_ref[...], g1_ref[...], be1_ref[...],
              W2_ref[...], b2_ref[...], g2_ref[...], be2_ref[...])
    x1_ref[...] = x1
    z = jnp.dot(x1, lW_ref[...], preferred_element_type=jnp.float32)
    z = z + lb_ref[...]
    Z_ref[...] = z0_ref[...] + z
    out_ref[...] = out0_ref[...] + _segmax_t(z.T, bT_ref[...], G)


def _sc_agg(x0, edge_index, zeros):
    N, D = x0.shape
    E = edge_index.shape[1]
    NW = 32
    EPW = E // NW
    K = 80
    NCH = EPW // K
    NG = 5
    GR = NCH // NG
    RPT = (N // 16) & ~7
    TAIL = N - 16 * RPT

    eidx5 = edge_index.reshape(2, NW, NG, GR, K)

    mesh = plsc.VectorSubcoreMesh(core_axis_name="c", subcore_axis_name="s")

    @functools.partial(
        pl.kernel,
        out_type=jax.ShapeDtypeStruct((2 * N, D), jnp.float32),
        mesh=mesh,
        scratch_types=[
            pltpu.VMEM((GR, K), jnp.int32),
            pltpu.VMEM((GR, K), jnp.int32),
            pltpu.VMEM((K, D), jnp.float32),
            pltpu.VMEM((K, D), jnp.float32),
            pltpu.VMEM((K, D), jnp.float32),
            pltpu.VMEM_SHARED((N, D), jnp.float32),
            pltpu.SemaphoreType.DMA,
            pltpu.SemaphoreType.DMA,
            pltpu.SemaphoreType.DMA,
            pltpu.SemaphoreType.DMA,
            pltpu.SemaphoreType.DMA,
            pltpu.SemaphoreType.DMA,
        ],
    )
    def k(x0_hbm, eidx_hbm, zero_hbm, out_hbm,
          sidx, didx, rows0, rows1, rows2, aggsp,
          gs0, gs1, gs2, ss0, ss1, ss2):
        rows = (rows0, rows1, rows2)
        gs = (gs0, gs1, gs2)
        ss = (ss0, ss1, ss2)
        c = lax.axis_index("c")
        s = lax.axis_index("s")
        wid = s * 2 + c
        @pl.when(c == 0)
        def _():
            pltpu.sync_copy(x0_hbm.at[pl.ds(s * RPT, RPT)],
                            aggsp.at[pl.ds(s * RPT, RPT)])

        @pl.when(c == 1)
        def _():
            pltpu.sync_copy(zero_hbm.at[pl.ds(s * RPT, RPT)],
                            aggsp.at[pl.ds(s * RPT, RPT)])

        @pl.when((s == 15) & (c == 0))
        def _():
            pltpu.sync_copy(x0_hbm.at[pl.ds(16 * RPT, TAIL)],
                            aggsp.at[pl.ds(16 * RPT, TAIL)])

        @pl.when((s == 15) & (c == 1))
        def _():
            pltpu.sync_copy(zero_hbm.at[pl.ds(16 * RPT, TAIL)],
                            aggsp.at[pl.ds(16 * RPT, TAIL)])

        plsc.subcore_barrier()

        def group(g, carry):
            pltpu.sync_copy(eidx_hbm.at[0, wid, g], sidx)
            pltpu.sync_copy(eidx_hbm.at[1, wid, g], didx)
            gd = [None] * GR
            sd = [None] * GR
            gd[0] = pltpu.async_copy(x0_hbm.at[sidx.at[0]], rows[0], gs[0])
            gd[1] = pltpu.async_copy(x0_hbm.at[sidx.at[1]], rows[1], gs[1])
            for i in range(GR):
                u = i % 3
                gd[i].wait()
                sd[i] = pltpu.async_copy(rows[u], aggsp.at[didx.at[i]],
                                         ss[u], add=True)
                if i + 2 < GR:
                    v = (i + 2) % 3
                    if i >= 1:
                        sd[i - 1].wait()
                    gd[i + 2] = pltpu.async_copy(x0_hbm.at[sidx.at[i + 2]],
                                                 rows[v], gs[v])
            sd[GR - 3].wait()
            sd[GR - 2].wait()
            sd[GR - 1].wait()
            return carry

        lax.fori_loop(0, NG, group, 0)

        plsc.subcore_barrier()
        pltpu.sync_copy(aggsp.at[pl.ds(s * RPT, RPT)],
                        out_hbm.at[pl.ds(c * N + s * RPT, RPT)])

        @pl.when(s == 15)
        def _():
            pltpu.sync_copy(aggsp.at[pl.ds(16 * RPT, TAIL)],
                            out_hbm.at[pl.ds(c * N + 16 * RPT, TAIL)])

    return k(x0, eidx5, zeros)


def kernel(x, fW1, fb1, fg1, fbe1, fW2, fb2, fg2, fbe2, l0W, l0b,
           cW1, cb1, cg1, cbe1, cW2, cb2, cg2, cbe2, l1W, l1b,
           edge_index, batch):
    N, D = x.shape
    H = fW1.shape[1]
    T = l0W.shape[1]
    G = 64

    bT = batch.reshape(1, N)
    r1 = lambda v: v.reshape(1, -1)

    x0 = pl.pallas_call(
        _mlp_body,
        out_shape=jax.ShapeDtypeStruct((N, H), jnp.float32),
    )(x, fW1, r1(fb1), r1(fg1), r1(fbe1), fW2, r1(fb2), r1(fg2), r1(fbe2))

    zeros = jnp.zeros((N, H), jnp.float32)
    aggp = _sc_agg(x0, edge_index, zeros)

    z0, out0 = pl.pallas_call(
        _head0_body,
        out_shape=(
            jax.ShapeDtypeStruct((N, T), jnp.float32),
            jax.ShapeDtypeStruct((G, T), jnp.float32),
        ),
    )(x0, l0W, r1(l0b), bT)

    out, Z, x1 = pl.pallas_call(
        _layer1_body,
        out_shape=(
            jax.ShapeDtypeStruct((G, T), jnp.float32),
            jax.ShapeDtypeStruct((N, T), jnp.float32),
            jax.ShapeDtypeStruct((N, H), jnp.float32),
        ),
    )(aggp, cW1, r1(cb1), r1(cg1), r1(cbe1),
      cW2, r1(cb2), r1(cg2), r1(cbe2), l1W, r1(l1b), bT, z0, out0)

    return (out, Z, x1)


# speedup vs baseline: 11.1049x; 1.0160x over previous
import functools

import jax
import jax.numpy as jnp
from jax import lax
from jax.experimental import pallas as pl
from jax.experimental.pallas import tpu as pltpu
from jax.experimental.pallas import tpu_sc as plsc

NEG = -1e30


def _bn_relu(h, g, be):
    m = jnp.mean(h, axis=0, keepdims=True)
    v = jnp.mean((h - m) ** 2, axis=0, keepdims=True)
    return jnp.maximum(g * (h - m) * lax.rsqrt(v + 1e-5) + be, 0.0)


def _mlp(x, W1, b1, g1, be1, W2, b2, g2, be2):
    h = jnp.dot(x, W1, preferred_element_type=jnp.float32)
    a = _bn_relu(h + b1, g1, be1)
    h2 = jnp.dot(a, W2, preferred_element_type=jnp.float32)
    return _bn_relu(h2 + b2, g2, be2)


def _segmax_t(zT, bT, G):
    cols = []
    for g in range(G):
        cols.append(jnp.max(jnp.where(bT == g, zT, NEG), axis=1, keepdims=True))
    return jnp.concatenate(cols, axis=1).T


def _mlp_body(x_ref, W1_ref, b1_ref, g1_ref, be1_ref, W2_ref, b2_ref,
              g2_ref, be2_ref, x0_ref):
    x0_ref[...] = _mlp(x_ref[...], W1_ref[...], b1_ref[...], g1_ref[...],
                       be1_ref[...], W2_ref[...], b2_ref[...], g2_ref[...],
                       be2_ref[...])


def _head0_body(x0_ref, lW_ref, lb_ref, bT_ref, z0_ref, out0_ref):
    G = out0_ref.shape[0]
    z = jnp.dot(x0_ref[...], lW_ref[...], preferred_element_type=jnp.float32)
    z = z + lb_ref[...]
    z0_ref[...] = z
    out0_ref[...] = _segmax_t(z.T, bT_ref[...], G)


def _layer1_body(pp_ref, W1_ref, b1_ref, g1_ref, be1_ref,
                 W2_ref, b2_ref, g2_ref, be2_ref, lW_ref, lb_ref, bT_ref,
                 z0_ref, out0_ref, out_ref, Z_ref, x1_ref):
    G = out_ref.shape[0]
    N = pp_ref.shape[0] // 2
    y = pp_ref[pl.ds(0, N), :] + pp_ref[pl.ds(N, N), :]
    x1 = _mlp(y, W1_ref[...], b1_ref[...], g1_ref[...], be1_ref[...],
              W2_ref[...], b2_ref[...], g2_ref[...], be2_ref[...])
    x1_ref[...] = x1
    z = jnp.dot(x1, lW_ref[...], preferred_element_type=jnp.float32)
    z = z + lb_ref[...]
    Z_ref[...] = z0_ref[...] + z
    out_ref[...] = out0_ref[...] + _segmax_t(z.T, bT_ref[...], G)


def _sc_agg(x0, edge_index):
    N, D = x0.shape
    E = edge_index.shape[1]
    NW = 32
    EPW = E // NW
    K = 80
    NCH = EPW // K
    NG = 5
    GR = NCH // NG
    RPT = (N // 16) & ~7
    TAIL = N - 16 * RPT

    eidx5 = edge_index.reshape(2, NW, NG, GR, K)

    mesh = plsc.VectorSubcoreMesh(core_axis_name="c", subcore_axis_name="s")

    @functools.partial(
        pl.kernel,
        out_type=jax.ShapeDtypeStruct((2 * N, D), jnp.float32),
        mesh=mesh,
        scratch_types=[
            pltpu.VMEM((GR, K), jnp.int32),
            pltpu.VMEM((GR, K), jnp.int32),
            pltpu.VMEM((K, D), jnp.float32),
            pltpu.VMEM((K, D), jnp.float32),
            pltpu.VMEM((K, D), jnp.float32),
            pltpu.VMEM_SHARED((N, D), jnp.float32),
            pltpu.SemaphoreType.DMA,
            pltpu.SemaphoreType.DMA,
            pltpu.SemaphoreType.DMA,
            pltpu.SemaphoreType.DMA,
            pltpu.SemaphoreType.DMA,
            pltpu.SemaphoreType.DMA,
        ],
    )
    def k(x0_hbm, eidx_hbm, out_hbm,
          sidx, didx, rows0, rows1, rows2, aggsp,
          gs0, gs1, gs2, ss0, ss1, ss2):
        rows = (rows0, rows1, rows2)
        gs = (gs0, gs1, gs2)
        ss = (ss0, ss1, ss2)
        c = lax.axis_index("c")
        s = lax.axis_index("s")
        wid = s * 2 + c
        @pl.when(c == 0)
        def _():
            pltpu.sync_copy(x0_hbm.at[pl.ds(s * RPT, RPT)],
                            aggsp.at[pl.ds(s * RPT, RPT)])

        @pl.when((s == 15) & (c == 0))
        def _():
            pltpu.sync_copy(x0_hbm.at[pl.ds(16 * RPT, TAIL)],
                            aggsp.at[pl.ds(16 * RPT, TAIL)])

        @pl.when(c == 1)
        def _():
            def zrow(r, carry):
                for l in range(D // 16):
                    rows0[r, pl.ds(l * 16, 16)] = jnp.zeros((16,), jnp.float32)
                return carry

            lax.fori_loop(0, K, zrow, 0)
            nfull, rem = RPT // K, RPT % K
            for j in range(nfull):
                pltpu.sync_copy(rows0, aggsp.at[pl.ds(s * RPT + j * K, K)])
            if rem:
                pltpu.sync_copy(rows0.at[pl.ds(0, rem)],
                                aggsp.at[pl.ds(s * RPT + nfull * K, rem)])

        @pl.when((s == 15) & (c == 1))
        def _():
            pltpu.sync_copy(rows0.at[pl.ds(0, TAIL)],
                            aggsp.at[pl.ds(16 * RPT, TAIL)])

        plsc.subcore_barrier()

        def group(g, carry):
            pltpu.sync_copy(eidx_hbm.at[0, wid, g], sidx)
            pltpu.sync_copy(eidx_hbm.at[1, wid, g], didx)
            gd = [None] * GR
            sd = [None] * GR
            gd[0] = pltpu.async_copy(x0_hbm.at[sidx.at[0]], rows[0], gs[0])
            gd[1] = pltpu.async_copy(x0_hbm.at[sidx.at[1]], rows[1], gs[1])
            for i in range(GR):
                u = i % 3
                gd[i].wait()
                sd[i] = pltpu.async_copy(rows[u], aggsp.at[didx.at[i]],
                                         ss[u], add=True)
                if i + 2 < GR:
                    v = (i + 2) % 3
                    if i >= 1:
                        sd[i - 1].wait()
                    gd[i + 2] = pltpu.async_copy(x0_hbm.at[sidx.at[i + 2]],
                                                 rows[v], gs[v])
            sd[GR - 3].wait()
            sd[GR - 2].wait()
            sd[GR - 1].wait()
            return carry

        lax.fori_loop(0, NG, group, 0)

        plsc.subcore_barrier()
        pltpu.sync_copy(aggsp.at[pl.ds(s * RPT, RPT)],
                        out_hbm.at[pl.ds(c * N + s * RPT, RPT)])

        @pl.when(s == 15)
        def _():
            pltpu.sync_copy(aggsp.at[pl.ds(16 * RPT, TAIL)],
                            out_hbm.at[pl.ds(c * N + 16 * RPT, TAIL)])

    return k(x0, eidx5)


def kernel(x, fW1, fb1, fg1, fbe1, fW2, fb2, fg2, fbe2, l0W, l0b,
           cW1, cb1, cg1, cbe1, cW2, cb2, cg2, cbe2, l1W, l1b,
           edge_index, batch):
    N, D = x.shape
    H = fW1.shape[1]
    T = l0W.shape[1]
    G = 64

    bT = batch.reshape(1, N)
    r1 = lambda v: v.reshape(1, -1)

    x0 = pl.pallas_call(
        _mlp_body,
        out_shape=jax.ShapeDtypeStruct((N, H), jnp.float32),
    )(x, fW1, r1(fb1), r1(fg1), r1(fbe1), fW2, r1(fb2), r1(fg2), r1(fbe2))

    aggp = _sc_agg(x0, edge_index)

    z0, out0 = pl.pallas_call(
        _head0_body,
        out_shape=(
            jax.ShapeDtypeStruct((N, T), jnp.float32),
            jax.ShapeDtypeStruct((G, T), jnp.float32),
        ),
    )(x0, l0W, r1(l0b), bT)

    out, Z, x1 = pl.pallas_call(
        _layer1_body,
        out_shape=(
            jax.ShapeDtypeStruct((G, T), jnp.float32),
            jax.ShapeDtypeStruct((N, T), jnp.float32),
            jax.ShapeDtypeStruct((N, H), jnp.float32),
        ),
    )(aggp, cW1, r1(cb1), r1(cg1), r1(cbe1),
      cW2, r1(cb2), r1(cg2), r1(cbe2), l1W, r1(l1b), bT, z0, out0)

    return (out, Z, x1)
